# baseline (speedup 1.0000x reference)
import functools

import jax
import jax.numpy as jnp
from jax.experimental import pallas as pl
from jax.experimental.pallas import tpu as pltpu


def _body(nxblk, x_ref, adj_ref, w1_ref, w2_ref, wo_ref, bo_ref, out_ref,
          s1, c_acc, h1_acc):
    i = pl.program_id(0)
    nsteps = pl.num_programs(0)
    ti = adj_ref.shape[0]
    tx = x_ref.shape[0]

    @pl.when(i < nxblk)
    def _prologue():
        s1[pl.ds(i * tx, tx), :] = jnp.dot(
            x_ref[...], w1_ref[...], preferred_element_type=jnp.float32)

    @pl.when(i >= nxblk)
    def _stream():
        r = i - nxblk
        blk = adj_ref[...]
        h1 = jnp.maximum(
            jnp.dot(blk, s1[...], preferred_element_type=jnp.float32), 0.0)
        h1_acc[pl.ds(r * ti, ti), :] = h1
        colsum = jnp.sum(blk, axis=0, keepdims=True)
        c_acc[...] = jnp.where(r == 0, colsum, c_acc[...] + colsum)

    @pl.when(i == nsteps - 1)
    def _epilogue():
        pooled = jnp.dot(c_acc[...], h1_acc[...],
                         preferred_element_type=jnp.float32)
        z = jnp.dot(pooled, w2_ref[...],
                    preferred_element_type=jnp.float32)
        o = jnp.dot(z, wo_ref[...],
                    preferred_element_type=jnp.float32) + bo_ref[...]
        out_ref[...] = jax.nn.sigmoid(o)


def kernel(x, adj, W1, W2, Wo, bo):
    n, d_in = x.shape
    h1_dim = W1.shape[1]
    h2_dim = W2.shape[1]

    ti = 200
    tx = 1000
    nblk = n // ti
    nxblk = n // tx

    out = pl.pallas_call(
        functools.partial(_body, nxblk),
        grid=(nblk + nxblk,),
        in_specs=[
            pl.BlockSpec((tx, d_in), lambda i: (jnp.minimum(i, nxblk - 1), 0)),
            pl.BlockSpec((ti, n), lambda i: (jnp.maximum(i - nxblk, 0), 0)),
            pl.BlockSpec((d_in, h1_dim), lambda i: (0, 0)),
            pl.BlockSpec((h1_dim, h2_dim), lambda i: (0, 0)),
            pl.BlockSpec((h2_dim, 1), lambda i: (0, 0)),
            pl.BlockSpec((1, 1), lambda i: (0, 0)),
        ],
        out_specs=pl.BlockSpec((1, 1), lambda i: (0, 0)),
        out_shape=jax.ShapeDtypeStruct((1, 1), jnp.float32),
        scratch_shapes=[
            pltpu.VMEM((n, h1_dim), jnp.float32),
            pltpu.VMEM((1, n), jnp.float32),
            pltpu.VMEM((n, h1_dim), jnp.float32),
        ],
        compiler_params=pltpu.CompilerParams(
            dimension_semantics=("arbitrary",)),
    )(x, adj, W1, W2, Wo, bo.reshape(1, 1))

    return out.reshape(1)


# speedup vs baseline: 1.0366x; 1.0366x over previous
import functools

import jax
import jax.numpy as jnp
from jax.experimental import pallas as pl
from jax.experimental.pallas import tpu as pltpu


def _body(nxblk, x_ref, adj_ref, w1_ref, w2_ref, wo_ref, bo_ref, out_ref,
          s1, c_acc, h1_acc):
    i = pl.program_id(0)
    nsteps = pl.num_programs(0)
    ti = adj_ref.shape[0]
    tx = x_ref.shape[0]

    @pl.when(i < nxblk)
    def _prologue():
        s1[pl.ds(i * tx, tx), :] = jnp.dot(
            x_ref[...], w1_ref[...], preferred_element_type=jnp.float32)

    @pl.when(i >= nxblk)
    def _stream():
        r = i - nxblk
        blk = adj_ref[...]
        h1 = jnp.maximum(
            jnp.dot(blk, s1[...], preferred_element_type=jnp.float32), 0.0)
        h1_acc[pl.ds(r * ti, ti), :] = h1
        colsum = jnp.sum(blk, axis=0, keepdims=True)
        c_acc[...] = jnp.where(r == 0, colsum, c_acc[...] + colsum)

    @pl.when(i == nsteps - 1)
    def _epilogue():
        pooled = jnp.dot(c_acc[...], h1_acc[...],
                         preferred_element_type=jnp.float32)
        z = jnp.dot(pooled, w2_ref[...],
                    preferred_element_type=jnp.float32)
        o = jnp.dot(z, wo_ref[...],
                    preferred_element_type=jnp.float32) + bo_ref[...]
        out_ref[...] = jax.nn.sigmoid(o)


def kernel(x, adj, W1, W2, Wo, bo):
    n, d_in = x.shape
    h1_dim = W1.shape[1]
    h2_dim = W2.shape[1]

    ti = 400
    tx = 2000
    nblk = n // ti
    nxblk = n // tx

    out = pl.pallas_call(
        functools.partial(_body, nxblk),
        grid=(nblk + nxblk,),
        in_specs=[
            pl.BlockSpec((tx, d_in), lambda i: (jnp.minimum(i, nxblk - 1), 0)),
            pl.BlockSpec((ti, n), lambda i: (jnp.maximum(i - nxblk, 0), 0)),
            pl.BlockSpec((d_in, h1_dim), lambda i: (0, 0)),
            pl.BlockSpec((h1_dim, h2_dim), lambda i: (0, 0)),
            pl.BlockSpec((h2_dim, 1), lambda i: (0, 0)),
            pl.BlockSpec((1, 1), lambda i: (0, 0)),
        ],
        out_specs=pl.BlockSpec((1, 1), lambda i: (0, 0)),
        out_shape=jax.ShapeDtypeStruct((1, 1), jnp.float32),
        scratch_shapes=[
            pltpu.VMEM((n, h1_dim), jnp.float32),
            pltpu.VMEM((1, n), jnp.float32),
            pltpu.VMEM((n, h1_dim), jnp.float32),
        ],
        compiler_params=pltpu.CompilerParams(
            dimension_semantics=("arbitrary",)),
    )(x, adj, W1, W2, Wo, bo.reshape(1, 1))

    return out.reshape(1)
